# baseline (speedup 1.0000x reference)
import functools

import jax
import jax.numpy as jnp
from jax import lax
from jax.experimental import pallas as pl
from jax.experimental.pallas import tpu as pltpu
from jax.experimental.pallas import tpu_sc as plsc

_N_NODES = 100000
_D = 128
_B = 65536

_info = plsc.get_sparse_core_info()
_NC, _NS = _info.num_cores, _info.num_subcores
_NW = _NC * _NS
_B_PER_W = _B // _NW
_CHUNK = 128
_N_CHUNKS = _B_PER_W // _CHUNK
_NBUF = 7
_DEPTH = 6

_mesh = plsc.VectorSubcoreMesh(core_axis_name="c", subcore_axis_name="s")


@functools.partial(
    pl.kernel,
    mesh=_mesh,
    out_type=jax.ShapeDtypeStruct((_B, _D), jnp.float32),
    scratch_types=[
        pltpu.VMEM((_B_PER_W,), jnp.int32),
        pltpu.VMEM((_NBUF, _CHUNK, _D), jnp.float32),
        pltpu.SemaphoreType.DMA((_NBUF,)),
        pltpu.SemaphoreType.DMA((_NBUF,)),
    ],
)
def _gather_rows(table_hbm, idx_hbm, out_hbm, idx_v, rows_v, gsems, osems):
    wid = lax.axis_index("s") * _NC + lax.axis_index("c")
    base = wid * _B_PER_W
    pltpu.sync_copy(idx_hbm.at[pl.ds(base, _B_PER_W)], idx_v)

    def gather_chunk(j, buf):
        return pltpu.async_copy(
            table_hbm.at[idx_v.at[pl.ds(j * _CHUNK, _CHUNK)]],
            rows_v.at[buf],
            gsems.at[buf],
        )

    def put_chunk(j, buf):
        return pltpu.async_copy(
            rows_v.at[buf],
            out_hbm.at[pl.ds(base + j * _CHUNK, _CHUNK)],
            osems.at[buf],
        )

    for j in range(_DEPTH):
        gather_chunk(j, j)

    def body(j, carry):
        buf = lax.rem(j, _NBUF)
        pltpu.make_async_copy(
            table_hbm.at[pl.ds(0, _CHUNK)], rows_v.at[buf], gsems.at[buf]
        ).wait()
        put_chunk(j, buf)
        nj = j + _DEPTH
        nbuf = lax.rem(nj, _NBUF)

        @pl.when((j >= _NBUF - _DEPTH) & (nj < _N_CHUNKS))
        def _():
            pltpu.make_async_copy(
                rows_v.at[nbuf], out_hbm.at[pl.ds(base, _CHUNK)], osems.at[nbuf]
            ).wait()

        @pl.when(nj < _N_CHUNKS)
        def _():
            gather_chunk(nj, nbuf)

        return carry

    lax.fori_loop(0, _N_CHUNKS, body, 0)
    for b in range(_NBUF):
        pltpu.make_async_copy(
            rows_v.at[b], out_hbm.at[pl.ds(base, _CHUNK)], osems.at[b]
        ).wait()


def kernel(node_features, time_w, time_b, source_nodes, timestamps,
           n_layers, n_neighbors):
    del time_w, time_b, timestamps, n_layers, n_neighbors
    return _gather_rows(node_features, source_nodes)
